# Initial kernel scaffold; baseline (speedup 1.0000x reference)
import jax
import jax.numpy as jnp
from jax.experimental import pallas as pl


def kernel(x, W_q_attn, W_q_pk, pk_keys, keys_table, values_table, W_out):
    raise NotImplementedError("write your pallas kernel here")


# speedup vs baseline: 3.8350x; 3.8350x over previous
import functools

import jax
import jax.numpy as jnp
from jax import lax
from jax.experimental import pallas as pl
from jax.experimental.pallas import tpu as pltpu
from jax.experimental.pallas import tpu_sc as plsc

N = 2048
DIM = 768
HEADS = 8
NUM_KV = 10000
NUM_KEYS = 100
TOPK = 8
DIM_KEY = 384

NB = 8
BN = N // NB
NEG = -3.0e38


def _pk_score_body(x_ref, w_ref, k_ref, o_ref):
    q = jnp.dot(x_ref[...], w_ref[0], preferred_element_type=jnp.float32)
    o_ref[0] = lax.dot_general(k_ref[0], q, (((1,), (1,)), ((), ())),
                               preferred_element_type=jnp.float32)


def _pk_score(x2, wpk, pkk):
    return pl.pallas_call(
        _pk_score_body,
        grid=(2 * HEADS, NB),
        in_specs=[
            pl.BlockSpec((BN, DIM), lambda ph, nb: (nb, 0)),
            pl.BlockSpec((1, DIM, DIM_KEY), lambda ph, nb: (ph, 0, 0)),
            pl.BlockSpec((1, NUM_KEYS, DIM_KEY), lambda ph, nb: (ph, 0, 0)),
        ],
        out_specs=pl.BlockSpec((1, NUM_KEYS, BN), lambda ph, nb: (ph, 0, nb)),
        out_shape=jax.ShapeDtypeStruct((2 * HEADS, NUM_KEYS, N), jnp.float32),
    )(x2, wpk, pkk)


def _top8T(s):
    iota = lax.broadcasted_iota(jnp.int32, s.shape, 0)
    vals, idxs = [], []
    for _ in range(TOPK):
        m = jnp.max(s, axis=0, keepdims=True)
        pos = jnp.min(jnp.where(s == m, iota, jnp.int32(2**30)), axis=0,
                      keepdims=True)
        vals.append(m)
        idxs.append(pos)
        s = jnp.where(iota == pos, NEG, s)
    return vals, idxs


def _pk_topk_body(sim_ref, sc_ref, ix_ref):
    for h in range(HEADS):
        v0, i0 = _top8T(sim_ref[0, h])
        v1, i1 = _top8T(sim_ref[1, h])
        v1c = jnp.concatenate(v1, axis=0)
        i1c = jnp.concatenate(i1, axis=0)
        s64 = jnp.concatenate([v0[i] + v1c for i in range(TOPK)], axis=0)
        i64 = jnp.concatenate([i0[i] + i1c * NUM_KEYS for i in range(TOPK)],
                              axis=0)
        iota = lax.broadcasted_iota(jnp.int32, s64.shape, 0)
        vals, kvs = [], []
        s = s64
        for _ in range(TOPK):
            m = jnp.max(s, axis=0, keepdims=True)
            pos = jnp.min(jnp.where(s == m, iota, jnp.int32(2**30)), axis=0,
                          keepdims=True)
            kv = jnp.sum(jnp.where(iota == pos, i64, 0), axis=0,
                         keepdims=True)
            vals.append(m)
            kvs.append(kv)
            s = jnp.where(iota == pos, NEG, s)
        fs = jnp.concatenate(vals, axis=0)
        fi = jnp.concatenate(kvs, axis=0)
        mm = jnp.max(fs, axis=0, keepdims=True)
        e = jnp.exp(fs - mm)
        sc_ref[h] = e / jnp.sum(e, axis=0, keepdims=True)
        ix_ref[h] = fi + h * NUM_KV


def _pk_topk(sim4):
    return pl.pallas_call(
        _pk_topk_body,
        grid=(NB,),
        in_specs=[pl.BlockSpec((2, HEADS, NUM_KEYS, BN),
                               lambda nb: (0, 0, 0, nb))],
        out_specs=[
            pl.BlockSpec((HEADS, TOPK, BN), lambda nb: (0, 0, nb)),
            pl.BlockSpec((HEADS, TOPK, BN), lambda nb: (0, 0, nb)),
        ],
        out_shape=[
            jax.ShapeDtypeStruct((HEADS, TOPK, N), jnp.float32),
            jax.ShapeDtypeStruct((HEADS, TOPK, N), jnp.int32),
        ],
    )(sim4)


def _q_proj_body(x_ref, w_ref, o_ref):
    o_ref[0] = jnp.dot(x_ref[...], w_ref[0],
                       preferred_element_type=jnp.float32) * (DIM ** -0.5)


def _q_proj(x2, wq):
    return pl.pallas_call(
        _q_proj_body,
        grid=(HEADS, NB),
        in_specs=[
            pl.BlockSpec((BN, DIM), lambda h, nb: (nb, 0)),
            pl.BlockSpec((1, DIM, DIM), lambda h, nb: (h, 0, 0)),
        ],
        out_specs=pl.BlockSpec((1, BN, DIM), lambda h, nb: (h, nb, 0)),
        out_shape=jax.ShapeDtypeStruct((HEADS, N, DIM), jnp.float32),
    )(x2, wq)


SC_NC = 2
SC_NS = 16
SC_L = 16
NW = SC_NC * SC_NS
SEG = HEADS * N
PER_W = SEG // NW
CSEG = 4
NCHUNK = PER_W // CSEG
CIDX = CSEG * TOPK
NVEC = DIM // SC_L


def _bag_accumulate(w_v, b, rows, acc):
    wvec = [w_v[b, pl.ds(g * SC_L, SC_L)] for g in range(CIDX // SC_L)]
    for s in range(CSEG):
        wsp = [jnp.full((SC_L,),
                        wvec[(s * TOPK + j) // SC_L][(s * TOPK + j) % SC_L],
                        jnp.float32)
               for j in range(TOPK)]

        def cbody(c, _, s=s, wsp=wsp):
            col = c * SC_L
            r = [wsp[j] * rows[b, s * TOPK + j, pl.ds(col, SC_L)]
                 for j in range(TOPK)]
            acc[s, pl.ds(col, SC_L)] = ((r[0] + r[1]) + (r[2] + r[3])) + \
                                       ((r[4] + r[5]) + (r[6] + r[7]))
            return 0

        lax.fori_loop(0, NVEC, cbody, 0, unroll=2)


def _sc_bag(idx_flat, w_flat, keys_table, values_table):
    mesh = plsc.VectorSubcoreMesh(core_axis_name="c", subcore_axis_name="s")
    npair = NCHUNK // 2

    @functools.partial(
        pl.kernel,
        out_type=[jax.ShapeDtypeStruct((SEG, DIM), jnp.float32),
                  jax.ShapeDtypeStruct((SEG, DIM), jnp.float32)],
        mesh=mesh,
        scratch_types=[
            pltpu.VMEM((2, CIDX), jnp.int32),
            pltpu.VMEM((2, CIDX), jnp.float32),
            pltpu.VMEM((2, CIDX, DIM), jnp.float32),
            pltpu.VMEM((2, CIDX, DIM), jnp.float32),
            pltpu.VMEM((CSEG, DIM), jnp.float32),
            pltpu.VMEM((CSEG, DIM), jnp.float32),
            pltpu.SemaphoreType.DMA,
            pltpu.SemaphoreType.DMA,
            pltpu.SemaphoreType.DMA,
            pltpu.SemaphoreType.DMA,
        ],
    )
    def bag(idx_hbm, w_hbm, keys_hbm, vals_hbm, kout, vout,
            idx_v, w_v, rk, rv, ak, av, sk0, sk1, sv0, sv1):
        wid = lax.axis_index("s") * SC_NC + lax.axis_index("c")
        base = wid * PER_W
        sems = ((sk0, sv0), (sk1, sv1))

        def issue(t, b):
            off = (base + t * CSEG) * TOPK
            pltpu.sync_copy(idx_hbm.at[pl.ds(off, CIDX)], idx_v.at[b])
            pltpu.sync_copy(w_hbm.at[pl.ds(off, CIDX)], w_v.at[b])
            pltpu.async_copy(keys_hbm.at[idx_v.at[b]], rk.at[b], sems[b][0])
            pltpu.async_copy(vals_hbm.at[idx_v.at[b]], rv.at[b], sems[b][1])

        def consume(t, b):
            seg0 = base + t * CSEG
            pltpu.make_async_copy(keys_hbm.at[idx_v.at[b]], rk.at[b],
                                  sems[b][0]).wait()
            _bag_accumulate(w_v, b, rk, ak)
            pltpu.make_async_copy(vals_hbm.at[idx_v.at[b]], rv.at[b],
                                  sems[b][1]).wait()
            _bag_accumulate(w_v, b, rv, av)
            pltpu.sync_copy(ak, kout.at[pl.ds(seg0, CSEG), :])
            pltpu.sync_copy(av, vout.at[pl.ds(seg0, CSEG), :])

        issue(0, 0)

        def pair(p, _):
            t0 = p * 2
            issue(t0 + 1, 1)
            consume(t0, 0)

            @pl.when(p < npair - 1)
            def _():
                issue(t0 + 2, 0)

            consume(t0 + 1, 1)
            return 0

        lax.fori_loop(0, npair, pair, 0)

    return bag(idx_flat, w_flat, keys_table, values_table)


def _flash_body(q_ref, k_ref, v_ref, o_ref):
    nb = pl.program_id(1)
    q = q_ref[0]

    def step(j, carry):
        m, l, acc = carry
        kj = k_ref[0, pl.ds(j * BN, BN), :]
        s = lax.dot_general(q, kj, (((1,), (1,)), ((), ())),
                            preferred_element_type=jnp.float32)
        rowi = lax.broadcasted_iota(jnp.int32, (BN, BN), 0)
        coli = lax.broadcasted_iota(jnp.int32, (BN, BN), 1)
        s = jnp.where((j == nb) & (coli > rowi), NEG, s)
        m_new = jnp.maximum(m, jnp.max(s, axis=1, keepdims=True))
        alpha = jnp.exp(m - m_new)
        p = jnp.exp(s - m_new)
        l_new = l * alpha + jnp.sum(p, axis=1, keepdims=True)
        vj = v_ref[0, pl.ds(j * BN, BN), :]
        acc_new = acc * alpha + jnp.dot(p, vj,
                                        preferred_element_type=jnp.float32)
        return m_new, l_new, acc_new

    m0 = jnp.full((BN, 1), NEG, jnp.float32)
    l0 = jnp.zeros((BN, 1), jnp.float32)
    a0 = jnp.zeros((BN, DIM), jnp.float32)
    _, l, acc = lax.fori_loop(0, nb + 1, step, (m0, l0, a0))
    o_ref[0] = acc / l


def _flash(q, kh, vh):
    return pl.pallas_call(
        _flash_body,
        grid=(HEADS, NB),
        in_specs=[
            pl.BlockSpec((1, BN, DIM), lambda h, nb: (h, nb, 0)),
            pl.BlockSpec((1, N, DIM), lambda h, nb: (h, 0, 0)),
            pl.BlockSpec((1, N, DIM), lambda h, nb: (h, 0, 0)),
        ],
        out_specs=pl.BlockSpec((1, BN, DIM), lambda h, nb: (h, nb, 0)),
        out_shape=jax.ShapeDtypeStruct((HEADS, N, DIM), jnp.float32),
    )(q, kh, vh)


def _proj_body(a_ref, w_ref, o_ref):
    @pl.when(pl.program_id(1) == 0)
    def _():
        o_ref[...] = jnp.zeros_like(o_ref)

    o_ref[...] += jnp.dot(a_ref[0], w_ref[0],
                          preferred_element_type=jnp.float32)


def _proj_out(ao, wout):
    return pl.pallas_call(
        _proj_body,
        grid=(NB, HEADS),
        in_specs=[
            pl.BlockSpec((1, BN, DIM), lambda nb, h: (h, nb, 0)),
            pl.BlockSpec((1, DIM, DIM), lambda nb, h: (h, 0, 0)),
        ],
        out_specs=pl.BlockSpec((BN, DIM), lambda nb, h: (nb, 0)),
        out_shape=jax.ShapeDtypeStruct((N, DIM), jnp.float32),
    )(ao, wout)


def kernel(x, W_q_attn, W_q_pk, pk_keys, keys_table, values_table, W_out):
    x2 = x[0]
    wq = W_q_attn.reshape(DIM, HEADS, DIM).transpose(1, 0, 2)
    wpk = W_q_pk.reshape(DIM, 2, HEADS, DIM_KEY).transpose(1, 2, 0, 3)
    wpk = wpk.reshape(2 * HEADS, DIM, DIM_KEY)
    pkk = pk_keys.transpose(0, 2, 1, 3).reshape(2 * HEADS, NUM_KEYS, DIM_KEY)
    wout = W_out.reshape(HEADS, DIM, DIM)

    sim = _pk_score(x2, wpk, pkk)
    scoresT, idxT = _pk_topk(sim.reshape(2, HEADS, NUM_KEYS, N))
    q = _q_proj(x2, wq)

    idx_flat = idxT.transpose(0, 2, 1).reshape(-1)
    w_flat = scoresT.transpose(0, 2, 1).reshape(-1)
    kbag, vbag = _sc_bag(idx_flat, w_flat, keys_table, values_table)
    kh = kbag.reshape(HEADS, N, DIM)
    vh = vbag.reshape(HEADS, N, DIM)

    ao = _flash(q, kh, vh)
    out = _proj_out(ao, wout)
    return out.reshape(1, N, DIM)
